# baseline (speedup 1.0000x reference)
import functools

import jax
import jax.numpy as jnp
from jax.experimental import pallas as pl
from jax.experimental.pallas import tpu as pltpu
from jax.experimental.pallas import tpu_sc as plsc

D = 4096
H = 2048
E = 64
TOP_K = 8

BM = 512
HB = BM // 2
TB = 128

_vector_mesh = plsc.VectorSubcoreMesh(core_axis_name="c", subcore_axis_name="s")


def _mlp_kernel(x_ref, w1_ref, b1_ref, w2_ref, b2_ref, probs_ref, ptT_ref):
    w1 = w1_ref[...]
    w2 = w2_ref[...]
    b1v = b1_ref[...]
    b2v = b2_ref[...]

    def half(lo):
        xb = x_ref[pl.ds(lo, HB), :].astype(jnp.bfloat16)
        acc = jnp.dot(xb, w1, preferred_element_type=jnp.float32)
        h = jnp.maximum(acc + b1v, 0.0)
        lg = jnp.dot(h.astype(jnp.bfloat16), w2,
                     preferred_element_type=jnp.float32)
        lt = (lg + b2v).T
        m = jnp.max(lt, axis=0, keepdims=True)
        e = jnp.exp(lt - m)
        return e / jnp.sum(e, axis=0, keepdims=True)

    pt0 = half(0)
    pt1 = half(HB)
    probs_ref[pl.ds(0, HB), :] = pt0.T
    probs_ref[pl.ds(HB, HB), :] = pt1.T
    ptT_ref[:, pl.ds(0, HB)] = pt0
    ptT_ref[:, pl.ds(HB, HB)] = pt1


def _mlp_chunk(x, w1b, b1r, w2b, b2r):
    M = x.shape[0]
    grid = (M // BM,)
    return pl.pallas_call(
        _mlp_kernel,
        grid=grid,
        in_specs=[
            pl.BlockSpec((BM, D), lambda m: (m, 0)),
            pl.BlockSpec((D, H), lambda m: (0, 0)),
            pl.BlockSpec((1, H), lambda m: (0, 0)),
            pl.BlockSpec((H, E), lambda m: (0, 0)),
            pl.BlockSpec((1, E), lambda m: (0, 0)),
        ],
        out_specs=[
            pl.BlockSpec((BM, E), lambda m: (m, 0)),
            pl.BlockSpec((E, BM), lambda m: (0, m)),
        ],
        out_shape=[
            jax.ShapeDtypeStruct((M, E), jnp.float32),
            jax.ShapeDtypeStruct((E, M), jnp.float32),
        ],
        compiler_params=pltpu.CompilerParams(
            dimension_semantics=("arbitrary",),
        ),
    )(x, w1b, b1r, w2b, b2r)


def _sc_topk(ptT):
    MT = ptT.shape[1]

    @pl.kernel(
        out_type=[jax.ShapeDtypeStruct((TOP_K, MT), jnp.int32),
                  jax.ShapeDtypeStruct((TOP_K, MT), jnp.float32)],
        mesh=_vector_mesh)
    def k(pt_hbm, idx_hbm, rw_hbm):
        def body(pt_vmem, idx_vmem, rw_vmem):
            @pl.loop(0, TB, step=16)
            def _(c):
                sl = pl.ds(c, 16)
                work = [pt_vmem.at[r, sl][...] for r in range(E)]
                vals, idxs = [], []
                for _t in range(TOP_K):
                    mx = work[0]
                    for r in range(1, E):
                        mx = jnp.maximum(mx, work[r])
                    cand = [jnp.where(work[r] == mx, jnp.int32(r), jnp.int32(E))
                            for r in range(E)]
                    ix = cand[0]
                    for r in range(1, E):
                        ix = jnp.minimum(ix, cand[r])
                    vals.append(mx)
                    idxs.append(ix)
                    work = [jnp.where(ix == jnp.int32(r), jnp.float32(-1.0),
                                      work[r]) for r in range(E)]
                m2 = vals[0]
                for t in range(1, TOP_K):
                    m2 = jnp.maximum(m2, vals[t])
                es = [jnp.exp(v - m2) for v in vals]
                s = es[0]
                for t in range(1, TOP_K):
                    s = s + es[t]
                for t in range(TOP_K):
                    idx_vmem.at[t, sl][...] = idxs[t]
                    rw_vmem.at[t, sl][...] = es[t] / s

        pltpu.emit_pipeline(
            body,
            grid=(MT // TB,),
            in_specs=[pl.BlockSpec((E, TB), lambda i: (0, i))],
            out_specs=[pl.BlockSpec((TOP_K, TB), lambda i: (0, i)),
                       pl.BlockSpec((TOP_K, TB), lambda i: (0, i))],
            core_axis_name=("c", "s"),
            dimension_semantics=(pltpu.PARALLEL,),
        )(pt_hbm, idx_hbm, rw_hbm)

    return k(ptT)


@functools.partial(jax.jit, static_argnames=())
def kernel(features, W1, b1, W2, b2):
    B, S, _ = features.shape
    M = B * S
    x = features.reshape(M, D)
    b1r = b1.reshape(1, H)
    b2r = b2.reshape(1, E)
    w1b = W1.astype(jnp.bfloat16)
    w2b = W2.astype(jnp.bfloat16)

    halves = jnp.split(x, 2, axis=0)
    probs_parts, idx_parts, rw_parts = [], [], []
    for xc in halves:
        probs_c, ptT_c = _mlp_chunk(xc, w1b, b1r, w2b, b2r)
        idxT_c, rwT_c = _sc_topk(ptT_c)
        probs_parts.append(probs_c)
        idx_parts.append(idxT_c.T)
        rw_parts.append(rwT_c.T)

    probs = jnp.concatenate(probs_parts, axis=0)
    idx = jnp.concatenate(idx_parts, axis=0)
    rw = jnp.concatenate(rw_parts, axis=0)
    return (probs.reshape(B, S, E),
            idx.reshape(B, S, TOP_K),
            rw.reshape(B, S, TOP_K))


# speedup vs baseline: 1.5056x; 1.5056x over previous
import functools

import jax
import jax.numpy as jnp
import numpy as np
from jax.experimental import pallas as pl
from jax.experimental.pallas import tpu as pltpu

D = 4096
H = 2048
E = 64
TOP_K = 8

BM = 512
HB = BM // 2


def _epilogue(logits, n):
    lt = logits.T
    m = jnp.max(lt, axis=0, keepdims=True)
    e = jnp.exp(lt - m)
    pt = e / jnp.sum(e, axis=0, keepdims=True)

    rows = jax.lax.broadcasted_iota(jnp.int32, (E, n), 0)
    work = pt
    vals = []
    idxs = []
    for _ in range(TOP_K):
        mx = jnp.max(work, axis=0, keepdims=True)
        cand = jnp.where(work == mx, rows, E)
        ix = jnp.min(cand, axis=0, keepdims=True)
        vals.append(mx)
        idxs.append(ix)
        work = jnp.where(rows == ix, -1.0, work)
    tkv = jnp.concatenate(vals, axis=0)
    tki = jnp.concatenate(idxs, axis=0)
    m2 = jnp.max(tkv, axis=0, keepdims=True)
    e2 = jnp.exp(tkv - m2)
    rw = e2 / jnp.sum(e2, axis=0, keepdims=True)
    return pt.T, tki.T, rw.T


def _router_kernel(x_ref, w1_ref, b1_ref, w2_ref, b2_ref,
                   probs_ref, idx_ref, rw_ref):
    w1 = w1_ref[...]
    w2 = w2_ref[...]
    b1v = b1_ref[...]
    b2v = b2_ref[...]

    def logits_half(lo):
        xb = x_ref[pl.ds(lo, HB), :].astype(jnp.bfloat16)
        acc = jnp.dot(xb, w1, preferred_element_type=jnp.float32)
        h = jnp.maximum(acc + b1v, 0.0)
        lg = jnp.dot(h.astype(jnp.bfloat16), w2,
                     preferred_element_type=jnp.float32)
        return lg + b2v

    lg0 = logits_half(0)
    lg1 = logits_half(HB)
    p0, i0, r0 = _epilogue(lg0, HB)
    p1, i1, r1 = _epilogue(lg1, HB)
    probs_ref[pl.ds(0, HB), :] = p0
    idx_ref[pl.ds(0, HB), :] = i0
    rw_ref[pl.ds(0, HB), :] = r0
    probs_ref[pl.ds(HB, HB), :] = p1
    idx_ref[pl.ds(HB, HB), :] = i1
    rw_ref[pl.ds(HB, HB), :] = r1


def _router_impl(features, W1, b1, W2, b2):
    B, S, _ = features.shape
    M = B * S
    x = features.reshape(M, D)
    b1r = b1.reshape(1, H)
    b2r = b2.reshape(1, E)
    w1b = W1.astype(jnp.bfloat16)
    w2b = W2.astype(jnp.bfloat16)

    grid = (M // BM,)
    probs, idx, rw = pl.pallas_call(
        _router_kernel,
        grid=grid,
        in_specs=[
            pl.BlockSpec((BM, D), lambda m: (m, 0)),
            pl.BlockSpec((D, H), lambda m: (0, 0)),
            pl.BlockSpec((1, H), lambda m: (0, 0)),
            pl.BlockSpec((H, E), lambda m: (0, 0)),
            pl.BlockSpec((1, E), lambda m: (0, 0)),
        ],
        out_specs=[
            pl.BlockSpec((BM, E), lambda m: (m, 0)),
            pl.BlockSpec((BM, TOP_K), lambda m: (m, 0)),
            pl.BlockSpec((BM, TOP_K), lambda m: (m, 0)),
        ],
        out_shape=[
            jax.ShapeDtypeStruct((M, E), jnp.float32),
            jax.ShapeDtypeStruct((M, TOP_K), jnp.int32),
            jax.ShapeDtypeStruct((M, TOP_K), jnp.float32),
        ],
        compiler_params=pltpu.CompilerParams(
            dimension_semantics=("arbitrary",),
        ),
    )(x, w1b, b1r, w2b, b2r)

    return (probs.reshape(B, S, E),
            idx.reshape(B, S, TOP_K),
            rw.reshape(B, S, TOP_K))


@functools.partial(jax.jit, static_argnames=())
def kernel(features, W1, b1, W2, b2):
    return _router_impl(features, W1, b1, W2, b2)
